# baseline (speedup 1.0000x reference)
import jax
import jax.numpy as jnp
from jax.experimental import pallas as pl
from jax.experimental.pallas import tpu as pltpu

_PAD = 64


def _router_kernel(x_ref, w1_ref, b1_ref, w2_ref, b2_ref,
                   probs_ref, idx_ref, val_ref):
    m = jnp.mean(x_ref[...], axis=2)
    z = jnp.maximum(
        jnp.dot(m, w1_ref[...], preferred_element_type=jnp.float32)
        + b1_ref[...], 0.0)
    logits = jnp.dot(z, w2_ref[...],
                     preferred_element_type=jnp.float32) + b2_ref[...]
    probs = jax.nn.softmax(logits, axis=1)
    E = probs.shape[1]
    col = jax.lax.broadcasted_iota(jnp.int32, probs.shape, 1)
    v1 = jnp.max(probs, axis=1, keepdims=True)
    i1 = jnp.min(jnp.where(probs == v1, col, E), axis=1, keepdims=True)
    masked = jnp.where(col == i1, -jnp.inf, probs)
    v2 = jnp.max(masked, axis=1, keepdims=True)
    i2 = jnp.min(jnp.where(masked == v2, col, E), axis=1, keepdims=True)
    probs_ref[...] = probs
    idx_ref[...] = jnp.concatenate([i1, i2], axis=1)
    val_ref[...] = jnp.concatenate([v1, v2], axis=1)


def _moe_kernel(idx_ref, val_ref, x_ref, w_ref, beta_ref, out_ref,
                xc_ref, xl_ref, xr_ref):
    b = pl.program_id(0)
    k = pl.program_id(1)
    C, HW = x_ref.shape[1], x_ref.shape[2]
    W = 56
    xc = x_ref[0]

    @pl.when(k == 0)
    def _build():
        xx = xc.T
        zrow = jnp.zeros((_PAD, C), jnp.float32)
        row = jax.lax.broadcasted_iota(jnp.int32, (HW, 1), 0) % W
        m_l = (row != 0).astype(jnp.float32)
        m_r = (row != W - 1).astype(jnp.float32)
        sh_l = jnp.concatenate([jnp.zeros((1, C), jnp.float32), xx[:-1, :]],
                               axis=0)
        sh_r = jnp.concatenate([xx[1:, :], jnp.zeros((1, C), jnp.float32)],
                               axis=0)
        for ref, mid in ((xc_ref, xx), (xl_ref, sh_l * m_l),
                         (xr_ref, sh_r * m_r)):
            ref[0:_PAD, :] = zrow
            ref[pl.ds(_PAD, HW), :] = mid
            ref[pl.ds(_PAD + HW, _PAD), :] = zrow

    rw = val_ref[b * 2 + k]
    acc = jnp.zeros((HW, C), jnp.float32)
    for t in range(9):
        dy, dx = t // 3 - 1, t % 3 - 1
        buf = (xl_ref, xc_ref, xr_ref)[dx + 1]
        sh = buf[pl.ds(_PAD + dy * W, HW), :]
        acc = acc + jnp.dot(sh, w_ref[0, t],
                            preferred_element_type=jnp.float32)
    o = jnp.maximum(acc + beta_ref[0], 0.0) * rw
    ot = o.T

    @pl.when(k == 0)
    def _init():
        out_ref[0] = xc + ot

    @pl.when(k != 0)
    def _accum():
        out_ref[0] = out_ref[0] + ot


def kernel(x, fc1_w, fc1_b, fc2_w, fc2_b, conv_w, bn_gamma, bn_beta):
    B, C, H, W = x.shape
    E, HID = fc2_w.shape[0], fc1_w.shape[0]
    HW = H * W
    K = 2
    x3 = x.reshape(B, C, HW)

    probs, idx2, val2 = pl.pallas_call(
        _router_kernel,
        out_shape=[
            jax.ShapeDtypeStruct((B, E), jnp.float32),
            jax.ShapeDtypeStruct((B, K), jnp.int32),
            jax.ShapeDtypeStruct((B, K), jnp.float32),
        ],
    )(x3, fc1_w.T, fc1_b.reshape(1, HID), fc2_w.T, fc2_b.reshape(1, E))

    eps = 1e-5
    scale = bn_gamma * (1.0 / jnp.sqrt(1.0 + eps))
    wt = conv_w.transpose(0, 3, 4, 2, 1).reshape(E, 9, C, C)
    wt = wt * scale[:, None, None, :]
    beta3 = bn_beta.reshape(E, 1, C)

    grid_spec = pltpu.PrefetchScalarGridSpec(
        num_scalar_prefetch=2,
        grid=(B, K),
        in_specs=[
            pl.BlockSpec((1, C, HW), lambda b, k, idx, val: (b, 0, 0)),
            pl.BlockSpec((1, 9, C, C),
                         lambda b, k, idx, val: (idx[b * 2 + k], 0, 0, 0)),
            pl.BlockSpec((1, 1, C),
                         lambda b, k, idx, val: (idx[b * 2 + k], 0, 0)),
        ],
        out_specs=pl.BlockSpec((1, C, HW), lambda b, k, idx, val: (b, 0, 0)),
        scratch_shapes=[pltpu.VMEM((2 * _PAD + HW, C), jnp.float32)] * 3,
    )
    out3 = pl.pallas_call(
        _moe_kernel,
        grid_spec=grid_spec,
        out_shape=jax.ShapeDtypeStruct((B, C, HW), jnp.float32),
    )(idx2.reshape(B * K), val2.reshape(B * K), x3, wt, beta3)

    return (out3.reshape(B, C, H, W), probs)


# speedup vs baseline: 1.4867x; 1.4867x over previous
import jax
import jax.numpy as jnp
from jax.experimental import pallas as pl
from jax.experimental.pallas import tpu as pltpu

_PAD = 64


def _router_kernel(x_ref, w1_ref, b1_ref, w2_ref, b2_ref,
                   probs_ref, idx_ref, val_ref):
    m = jnp.mean(x_ref[...], axis=1)
    z = jnp.maximum(
        jnp.dot(m, w1_ref[...], preferred_element_type=jnp.float32)
        + b1_ref[...], 0.0)
    logits = jnp.dot(z, w2_ref[...],
                     preferred_element_type=jnp.float32) + b2_ref[...]
    probs = jax.nn.softmax(logits, axis=1)
    E = probs.shape[1]
    col = jax.lax.broadcasted_iota(jnp.int32, probs.shape, 1)
    v1 = jnp.max(probs, axis=1, keepdims=True)
    i1 = jnp.min(jnp.where(probs == v1, col, E), axis=1, keepdims=True)
    masked = jnp.where(col == i1, -jnp.inf, probs)
    v2 = jnp.max(masked, axis=1, keepdims=True)
    i2 = jnp.min(jnp.where(masked == v2, col, E), axis=1, keepdims=True)
    probs_ref[...] = probs
    idx_ref[...] = jnp.concatenate([i1, i2], axis=1)
    val_ref[...] = jnp.concatenate([v1, v2], axis=1)


def _moe_kernel(idx_ref, val_ref, x_ref, w0_ref, w1_ref, beta0_ref,
                beta1_ref, out_ref, xc_ref, xl_ref, xr_ref):
    b = pl.program_id(0)
    HW, C = x_ref.shape[1], x_ref.shape[2]
    W = 56
    xx = x_ref[0]

    zrow = jnp.zeros((_PAD, C), jnp.float32)
    row = jax.lax.broadcasted_iota(jnp.int32, (HW, 1), 0) % W
    m_l = (row != 0).astype(jnp.float32)
    m_r = (row != W - 1).astype(jnp.float32)
    sh_l = jnp.concatenate([jnp.zeros((1, C), jnp.float32), xx[:-1, :]],
                           axis=0)
    sh_r = jnp.concatenate([xx[1:, :], jnp.zeros((1, C), jnp.float32)],
                           axis=0)
    for ref, mid in ((xc_ref, xx), (xl_ref, sh_l * m_l),
                     (xr_ref, sh_r * m_r)):
        ref[0:_PAD, :] = zrow
        ref[pl.ds(_PAD, HW), :] = mid
        ref[pl.ds(_PAD + HW, _PAD), :] = zrow

    rw0 = val_ref[b * 2]
    rw1 = val_ref[b * 2 + 1]
    acc0 = jnp.zeros((HW, C), jnp.float32)
    acc1 = jnp.zeros((HW, C), jnp.float32)
    for t in range(9):
        dy, dx = t // 3 - 1, t % 3 - 1
        buf = (xl_ref, xc_ref, xr_ref)[dx + 1]
        sh = buf[pl.ds(_PAD + dy * W, HW), :]
        acc0 = acc0 + jnp.dot(sh, w0_ref[0, t],
                              preferred_element_type=jnp.float32)
        acc1 = acc1 + jnp.dot(sh, w1_ref[0, t],
                              preferred_element_type=jnp.float32)
    o0 = jnp.maximum(acc0 + beta0_ref[0], 0.0) * rw0
    o1 = jnp.maximum(acc1 + beta1_ref[0], 0.0) * rw1
    out_ref[0] = xx + o0 + o1


def kernel(x, fc1_w, fc1_b, fc2_w, fc2_b, conv_w, bn_gamma, bn_beta):
    B, C, H, W = x.shape
    E, HID = fc2_w.shape[0], fc1_w.shape[0]
    HW = H * W
    K = 2
    xt = x.reshape(B, C, HW).transpose(0, 2, 1)

    probs, idx2, val2 = pl.pallas_call(
        _router_kernel,
        out_shape=[
            jax.ShapeDtypeStruct((B, E), jnp.float32),
            jax.ShapeDtypeStruct((B, K), jnp.int32),
            jax.ShapeDtypeStruct((B, K), jnp.float32),
        ],
    )(xt, fc1_w.T, fc1_b.reshape(1, HID), fc2_w.T, fc2_b.reshape(1, E))

    eps = 1e-5
    scale = bn_gamma * (1.0 / jnp.sqrt(1.0 + eps))
    wt = conv_w.transpose(0, 3, 4, 2, 1).reshape(E, 9, C, C)
    wt = wt * scale[:, None, None, :]
    beta3 = bn_beta.reshape(E, 1, C)

    grid_spec = pltpu.PrefetchScalarGridSpec(
        num_scalar_prefetch=2,
        grid=(B,),
        in_specs=[
            pl.BlockSpec((1, HW, C), lambda b, idx, val: (b, 0, 0)),
            pl.BlockSpec((1, 9, C, C),
                         lambda b, idx, val: (idx[b * 2], 0, 0, 0)),
            pl.BlockSpec((1, 9, C, C),
                         lambda b, idx, val: (idx[b * 2 + 1], 0, 0, 0)),
            pl.BlockSpec((1, 1, C),
                         lambda b, idx, val: (idx[b * 2], 0, 0)),
            pl.BlockSpec((1, 1, C),
                         lambda b, idx, val: (idx[b * 2 + 1], 0, 0)),
        ],
        out_specs=pl.BlockSpec((1, HW, C), lambda b, idx, val: (b, 0, 0)),
        scratch_shapes=[pltpu.VMEM((2 * _PAD + HW, C), jnp.float32)] * 3,
    )
    out_t = pl.pallas_call(
        _moe_kernel,
        grid_spec=grid_spec,
        out_shape=jax.ShapeDtypeStruct((B, HW, C), jnp.float32),
    )(idx2.reshape(B * K), val2.reshape(B * K), xt, wt, wt, beta3, beta3)

    out = out_t.transpose(0, 2, 1).reshape(B, C, H, W)
    return (out, probs)
